# Initial kernel scaffold; baseline (speedup 1.0000x reference)
import functools

import jax
import jax.numpy as jnp
from jax import lax
from jax.experimental import pallas as pl
from jax.experimental.pallas import tpu as pltpu
from jax.experimental.pallas import tpu_sc as plsc

_V = 1000
_D = 128
_L = 200
_B = 4096

_N = _B * _L
_NW = 32
_TOK_W = _N // _NW
_CH = 100
_NCH = _TOK_W // _CH
_LANES = 16


def _pos_encoding():
    even_i = jnp.arange(0, _D, 2).astype(jnp.float32)
    denominator = jnp.power(10000.0, even_i / _D)
    position = jnp.arange(_L).reshape(_L, 1).astype(jnp.float32)
    even_pos = jnp.sin(position / denominator)
    odd_pos = jnp.cos(position / denominator)
    return jnp.stack([even_pos, odd_pos], axis=2).reshape(_L, _D)


_mesh = plsc.VectorSubcoreMesh(core_axis_name="c", subcore_axis_name="s")


@functools.partial(
    pl.kernel,
    out_type=jax.ShapeDtypeStruct((_N, _D), jnp.float32),
    mesh=_mesh,
    scratch_types=[
        pltpu.VMEM((_NCH, _CH), jnp.int32),
        pltpu.VMEM((2, _CH, _D), jnp.float32),
        pltpu.VMEM((_CH, _D), jnp.float32),
        pltpu.SemaphoreType.DMA,
    ],
)
def _emb_kernel(table_hbm, idx_hbm, pos_hbm, out_hbm, idx_v, pos_v, rows_v, sem):
    wid = lax.axis_index("s") * 2 + lax.axis_index("c")
    pltpu.sync_copy(idx_hbm.at[wid], idx_v)
    pltpu.sync_copy(pos_hbm, pos_v)
    base = wid * _TOK_W

    @pl.loop(0, _NCH)
    def _chunk(c):
        pltpu.async_copy(table_hbm.at[idx_v.at[c]], rows_v, sem).wait()
        half = lax.rem(c, 2)

        @pl.loop(0, _CH)
        def _row(r):
            for d in range(_D // _LANES):
                sl = pl.ds(d * _LANES, _LANES)
                plsc.addupdate(rows_v.at[r, sl], pos_v[half, r, sl])

        pltpu.sync_copy(rows_v, out_hbm.at[pl.ds(base + c * _CH, _CH)])


def kernel(x, start_token, end_token, embedding_table):
    idx = x.reshape(_NW, _NCH, _CH).astype(jnp.int32)
    pos = _pos_encoding().reshape(2, _CH, _D)
    out = _emb_kernel(embedding_table, idx, pos)
    return out.reshape(_B, _L, _D)


# speedup vs baseline: 2.1900x; 2.1900x over previous
import functools

import jax
import jax.numpy as jnp
from jax import lax
from jax.experimental import pallas as pl
from jax.experimental.pallas import tpu as pltpu
from jax.experimental.pallas import tpu_sc as plsc

_V = 1000
_D = 128
_L = 200
_B = 4096

_N = _B * _L
_NW = 32
_TOK_W = _N // _NW
_CH = 128
_NCH = _TOK_W // _CH
_LANES = 16


def _pos_encoding():
    even_i = jnp.arange(0, _D, 2).astype(jnp.float32)
    denominator = jnp.power(10000.0, even_i / _D)
    position = jnp.arange(_L).reshape(_L, 1).astype(jnp.float32)
    even_pos = jnp.sin(position / denominator)
    odd_pos = jnp.cos(position / denominator)
    return jnp.stack([even_pos, odd_pos], axis=2).reshape(_L, _D)


_mesh = plsc.VectorSubcoreMesh(core_axis_name="c", subcore_axis_name="s")


@functools.partial(
    pl.kernel,
    out_type=jax.ShapeDtypeStruct((_N, _D), jnp.float32),
    mesh=_mesh,
    scratch_types=[
        pltpu.VMEM((_NCH, _CH), jnp.int32),
        pltpu.VMEM((2 * _L, _D), jnp.float32),
        pltpu.VMEM((_CH, _D), jnp.float32),
        pltpu.SemaphoreType.DMA,
    ],
)
def _emb_kernel(table_hbm, idx_hbm, pos_hbm, out_hbm, idx_v, pos_v, rows_v, sem):
    wid = lax.axis_index("s") * 2 + lax.axis_index("c")
    pltpu.sync_copy(idx_hbm.at[wid], idx_v)
    pltpu.sync_copy(pos_hbm, pos_v)
    base = wid * _TOK_W

    @pl.loop(0, _NCH)
    def _chunk(c):
        pltpu.async_copy(table_hbm.at[idx_v.at[c]], rows_v, sem).wait()
        p0 = lax.rem(c * _CH, _L)

        @pl.loop(0, _CH)
        def _row(r):
            for d in range(_D // _LANES):
                sl = pl.ds(d * _LANES, _LANES)
                plsc.addupdate(rows_v.at[r, sl], pos_v[p0 + r, sl])

        pltpu.sync_copy(rows_v, out_hbm.at[pl.ds(base + c * _CH, _CH)])


def kernel(x, start_token, end_token, embedding_table):
    idx = x.reshape(_NW, _NCH, _CH).astype(jnp.int32)
    pos = _pos_encoding()
    pos2 = jnp.concatenate([pos, pos], axis=0)
    out = _emb_kernel(embedding_table, idx, pos2)
    return out.reshape(_B, _L, _D)


# speedup vs baseline: 2.9980x; 1.3690x over previous
import functools

import jax
import jax.numpy as jnp
from jax import lax
from jax.experimental import pallas as pl
from jax.experimental.pallas import tpu as pltpu
from jax.experimental.pallas import tpu_sc as plsc

_V = 1000
_D = 128
_L = 200
_B = 4096

_N = _B * _L
_NW = 32
_TOK_W = _N // _NW
_CH = 128
_NCH = _TOK_W // _CH
_LANES = 16


def _pos_encoding():
    even_i = jnp.arange(0, _D, 2).astype(jnp.float32)
    denominator = jnp.power(10000.0, even_i / _D)
    position = jnp.arange(_L).reshape(_L, 1).astype(jnp.float32)
    even_pos = jnp.sin(position / denominator)
    odd_pos = jnp.cos(position / denominator)
    return jnp.stack([even_pos, odd_pos], axis=2).reshape(_L, _D)


_mesh = plsc.VectorSubcoreMesh(core_axis_name="c", subcore_axis_name="s")


@functools.partial(
    pl.kernel,
    out_type=jax.ShapeDtypeStruct((_N, _D), jnp.float32),
    mesh=_mesh,
    scratch_types=[
        pltpu.VMEM((_NCH, _CH), jnp.int32),
        pltpu.VMEM((2 * _L, _D), jnp.float32),
        pltpu.VMEM((2, _CH, _D), jnp.float32),
        pltpu.SemaphoreType.DMA,
        pltpu.SemaphoreType.DMA,
        pltpu.SemaphoreType.DMA,
        pltpu.SemaphoreType.DMA,
    ],
)
def _emb_kernel(table_hbm, idx_hbm, pos_hbm, out_hbm, idx_v, pos_v, rows_v,
                gsem0, gsem1, ssem0, ssem1):
    gsem = (gsem0, gsem1)
    ssem = (ssem0, ssem1)
    wid = lax.axis_index("s") * 2 + lax.axis_index("c")
    pltpu.sync_copy(idx_hbm.at[wid], idx_v)
    pltpu.sync_copy(pos_hbm, pos_v)
    base = wid * _TOK_W

    pltpu.async_copy(table_hbm.at[idx_v.at[0]], rows_v.at[0], gsem[0])

    @pl.loop(0, _NCH, step=2)
    def _chunk(c0):
        for b in range(2):
            c = c0 + b
            nb = 1 - b

            @pl.when(c > 0)
            def _():
                pltpu.make_async_copy(
                    rows_v.at[nb],
                    out_hbm.at[pl.ds(base + (c - 1) * _CH, _CH)],
                    ssem[nb],
                ).wait()

            @pl.when(c + 1 < _NCH)
            def _():
                pltpu.async_copy(table_hbm.at[idx_v.at[c + 1]], rows_v.at[nb],
                                 gsem[nb])

            pltpu.make_async_copy(table_hbm.at[idx_v.at[c]], rows_v.at[b],
                                  gsem[b]).wait()
            p0 = lax.rem(c * _CH, _L)

            @pl.loop(0, _CH, unroll=4)
            def _row(r):
                for d in range(_D // _LANES):
                    sl = pl.ds(d * _LANES, _LANES)
                    plsc.addupdate(rows_v.at[b, r, sl], pos_v[p0 + r, sl])

            pltpu.async_copy(rows_v.at[b],
                             out_hbm.at[pl.ds(base + c * _CH, _CH)], ssem[b])

    pltpu.make_async_copy(rows_v.at[1],
                          out_hbm.at[pl.ds(base + (_NCH - 1) * _CH, _CH)],
                          ssem[1]).wait()


def kernel(x, start_token, end_token, embedding_table):
    idx = x.reshape(_NW, _NCH, _CH).astype(jnp.int32)
    pos = _pos_encoding()
    pos2 = jnp.concatenate([pos, pos], axis=0)
    out = _emb_kernel(embedding_table, idx, pos2)
    return out.reshape(_B, _L, _D)


# speedup vs baseline: 3.4479x; 1.1501x over previous
import functools

import jax
import jax.numpy as jnp
from jax import lax
from jax.experimental import pallas as pl
from jax.experimental.pallas import tpu as pltpu
from jax.experimental.pallas import tpu_sc as plsc

_V = 1000
_D = 128
_L = 200
_B = 4096

_N = _B * _L
_NW = 32
_TOK_W = _N // _NW
_CH = 128
_NCH = _TOK_W // _CH
_NBUF = 4
_LANES = 16


def _pos_encoding():
    even_i = jnp.arange(0, _D, 2).astype(jnp.float32)
    denominator = jnp.power(10000.0, even_i / _D)
    position = jnp.arange(_L).reshape(_L, 1).astype(jnp.float32)
    even_pos = jnp.sin(position / denominator)
    odd_pos = jnp.cos(position / denominator)
    return jnp.stack([even_pos, odd_pos], axis=2).reshape(_L, _D)


_mesh = plsc.VectorSubcoreMesh(core_axis_name="c", subcore_axis_name="s")


@functools.partial(
    pl.kernel,
    out_type=jax.ShapeDtypeStruct((_N, _D), jnp.float32),
    mesh=_mesh,
    scratch_types=[
        pltpu.VMEM((_NCH, _CH), jnp.int32),
        pltpu.VMEM((_L, _D), jnp.float32),
        pltpu.VMEM((_NBUF, _CH, _D), jnp.float32),
        [pltpu.SemaphoreType.DMA] * _NBUF,
        [pltpu.SemaphoreType.DMA] * _NBUF,
    ],
)
def _emb_kernel(table_hbm, idx_hbm, pos_hbm, out_hbm, idx_v, pos_v, rows_v,
                gsem, ssem):
    wid = lax.axis_index("s") * 2 + lax.axis_index("c")
    pltpu.sync_copy(idx_hbm.at[wid], idx_v)
    pltpu.sync_copy(pos_hbm, pos_v)
    base = wid * _TOK_W

    pltpu.async_copy(table_hbm.at[idx_v.at[0]], rows_v.at[0], gsem[0])
    pltpu.async_copy(table_hbm.at[idx_v.at[1]], rows_v.at[1], gsem[1])

    @pl.loop(0, _NCH, step=_NBUF)
    def _chunk(c0):
        for b in range(_NBUF):
            c = c0 + b
            tb = (b + 2) % _NBUF

            @pl.when(c >= 2)
            def _():
                pltpu.make_async_copy(
                    rows_v.at[tb],
                    out_hbm.at[pl.ds(base + (c - 2) * _CH, _CH)],
                    ssem[tb],
                ).wait()

            @pl.when(c + 2 < _NCH)
            def _():
                pltpu.async_copy(table_hbm.at[idx_v.at[c + 2]], rows_v.at[tb],
                                 gsem[tb])

            pltpu.make_async_copy(table_hbm.at[idx_v.at[c]], rows_v.at[b],
                                  gsem[b]).wait()

            p0 = lax.rem(c * _CH, _L)
            n1 = jnp.minimum(_L - p0, _CH)

            @pl.loop(0, n1)
            def _row_lo(r):
                for d in range(_D // _LANES):
                    sl = pl.ds(d * _LANES, _LANES)
                    plsc.addupdate(rows_v.at[b, r, sl], pos_v[p0 + r, sl])

            @pl.loop(n1, _CH)
            def _row_hi(r):
                for d in range(_D // _LANES):
                    sl = pl.ds(d * _LANES, _LANES)
                    plsc.addupdate(rows_v.at[b, r, sl], pos_v[p0 + r - _L, sl])

            pltpu.async_copy(rows_v.at[b],
                             out_hbm.at[pl.ds(base + c * _CH, _CH)], ssem[b])

    for c in (_NCH - 2, _NCH - 1):
        pltpu.make_async_copy(rows_v.at[c % _NBUF],
                              out_hbm.at[pl.ds(base + c * _CH, _CH)],
                              ssem[c % _NBUF]).wait()


def kernel(x, start_token, end_token, embedding_table):
    idx = x.reshape(_NW, _NCH, _CH).astype(jnp.int32)
    pos = _pos_encoding()
    out = _emb_kernel(embedding_table, idx, pos)
    return out.reshape(_B, _L, _D)


# speedup vs baseline: 3.4623x; 1.0042x over previous
import functools

import jax
import jax.numpy as jnp
from jax import lax
from jax.experimental import pallas as pl
from jax.experimental.pallas import tpu as pltpu
from jax.experimental.pallas import tpu_sc as plsc

_V = 1000
_D = 128
_L = 200
_B = 4096

_N = _B * _L
_NW = 32
_TOK_W = _N // _NW
_CH = 128
_NCH = _TOK_W // _CH
_NBUF = 4
_LANES = 16


def _pos_encoding():
    even_i = jnp.arange(0, _D, 2).astype(jnp.float32)
    denominator = jnp.power(10000.0, even_i / _D)
    position = jnp.arange(_L).reshape(_L, 1).astype(jnp.float32)
    even_pos = jnp.sin(position / denominator)
    odd_pos = jnp.cos(position / denominator)
    return jnp.stack([even_pos, odd_pos], axis=2).reshape(_L, _D)


_mesh = plsc.VectorSubcoreMesh(core_axis_name="c", subcore_axis_name="s")


@functools.partial(
    pl.kernel,
    out_type=jax.ShapeDtypeStruct((_N, _D), jnp.float32),
    mesh=_mesh,
    scratch_types=[
        pltpu.VMEM((_NCH, _CH), jnp.int32),
        pltpu.VMEM((_L, _D), jnp.float32),
        pltpu.VMEM((_NBUF, _CH, _D), jnp.float32),
        pltpu.VMEM_SHARED((_V, _D), jnp.float32),
        [pltpu.SemaphoreType.DMA] * _NBUF,
        [pltpu.SemaphoreType.DMA] * _NBUF,
    ],
)
def _emb_kernel(table_hbm, idx_hbm, pos_hbm, out_hbm, idx_v, pos_v, rows_v,
                table_sh, gsem, ssem):
    sid = lax.axis_index("s")
    wid = sid * 2 + lax.axis_index("c")

    @pl.when(sid == 0)
    def _():
        pltpu.sync_copy(table_hbm, table_sh)

    pltpu.sync_copy(idx_hbm.at[wid], idx_v)
    pltpu.sync_copy(pos_hbm, pos_v)
    plsc.subcore_barrier()
    base = wid * _TOK_W

    pltpu.async_copy(table_sh.at[idx_v.at[0]], rows_v.at[0], gsem[0])
    pltpu.async_copy(table_sh.at[idx_v.at[1]], rows_v.at[1], gsem[1])

    @pl.loop(0, _NCH, step=_NBUF)
    def _chunk(c0):
        for b in range(_NBUF):
            c = c0 + b
            tb = (b + 2) % _NBUF

            @pl.when(c >= 2)
            def _():
                pltpu.make_async_copy(
                    rows_v.at[tb],
                    out_hbm.at[pl.ds(base + (c - 2) * _CH, _CH)],
                    ssem[tb],
                ).wait()

            @pl.when(c + 2 < _NCH)
            def _():
                pltpu.async_copy(table_sh.at[idx_v.at[c + 2]], rows_v.at[tb],
                                 gsem[tb])

            pltpu.make_async_copy(table_sh.at[idx_v.at[c]], rows_v.at[b],
                                  gsem[b]).wait()

            p0 = lax.rem(c * _CH, _L)
            n1 = jnp.minimum(_L - p0, _CH)

            @pl.loop(0, n1)
            def _row_lo(r):
                for d in range(_D // _LANES):
                    sl = pl.ds(d * _LANES, _LANES)
                    plsc.addupdate(rows_v.at[b, r, sl], pos_v[p0 + r, sl])

            @pl.loop(n1, _CH)
            def _row_hi(r):
                for d in range(_D // _LANES):
                    sl = pl.ds(d * _LANES, _LANES)
                    plsc.addupdate(rows_v.at[b, r, sl], pos_v[p0 + r - _L, sl])

            pltpu.async_copy(rows_v.at[b],
                             out_hbm.at[pl.ds(base + c * _CH, _CH)], ssem[b])

    for c in (_NCH - 2, _NCH - 1):
        pltpu.make_async_copy(rows_v.at[c % _NBUF],
                              out_hbm.at[pl.ds(base + c * _CH, _CH)],
                              ssem[c % _NBUF]).wait()


def kernel(x, start_token, end_token, embedding_table):
    idx = x.reshape(_NW, _NCH, _CH).astype(jnp.int32)
    pos = _pos_encoding()
    out = _emb_kernel(embedding_table, idx, pos)
    return out.reshape(_B, _L, _D)


# speedup vs baseline: 4.0054x; 1.1569x over previous
import functools

import jax
import jax.numpy as jnp
from jax import lax
from jax.experimental import pallas as pl
from jax.experimental.pallas import tpu as pltpu
from jax.experimental.pallas import tpu_sc as plsc

_V = 1000
_D = 128
_L = 200
_B = 4096

_N = _B * _L
_NW = 32
_TOK_W = _N // _NW
_CH = 128
_NCH = _TOK_W // _CH
_NBUF = 4
_NPH = 25
_LANES = 16


def _pos_encoding():
    even_i = jnp.arange(0, _D, 2).astype(jnp.float32)
    denominator = jnp.power(10000.0, even_i / _D)
    position = jnp.arange(_L).reshape(_L, 1).astype(jnp.float32)
    even_pos = jnp.sin(position / denominator)
    odd_pos = jnp.cos(position / denominator)
    return jnp.stack([even_pos, odd_pos], axis=2).reshape(_L, _D)


_mesh = plsc.VectorSubcoreMesh(core_axis_name="c", subcore_axis_name="s")


@functools.partial(
    pl.kernel,
    out_type=jax.ShapeDtypeStruct((_N, _D), jnp.float32),
    mesh=_mesh,
    scratch_types=[
        pltpu.VMEM((_NCH, _CH), jnp.int32),
        pltpu.VMEM((_L, _D), jnp.float32),
        pltpu.VMEM((_NBUF, _CH, _D), jnp.float32),
        pltpu.VMEM_SHARED((_V, _D), jnp.float32),
        [pltpu.SemaphoreType.DMA] * _NBUF,
        [pltpu.SemaphoreType.DMA] * _NBUF,
    ],
)
def _emb_kernel(table_hbm, idx_hbm, pos_hbm, out_hbm, idx_v, pos_v, rows_v,
                table_sh, gsem, ssem):
    sid = lax.axis_index("s")
    wid = sid * 2 + lax.axis_index("c")

    @pl.when(sid == 0)
    def _():
        pltpu.sync_copy(table_hbm, table_sh)

    pltpu.sync_copy(idx_hbm.at[wid], idx_v)
    pltpu.sync_copy(pos_hbm, pos_v)
    plsc.subcore_barrier()
    base = wid * _TOK_W

    pltpu.async_copy(table_sh.at[idx_v.at[0]], rows_v.at[0], gsem[0])
    pltpu.async_copy(table_sh.at[idx_v.at[1]], rows_v.at[1], gsem[1])

    @pl.loop(0, _NCH, step=_NBUF)
    def _chunk(c0):
        for b in range(_NBUF):
            c = c0 + b
            gb = (b + 2) % _NBUF

            @pl.when(c >= 2)
            def _():
                pltpu.make_async_copy(
                    rows_v.at[gb],
                    out_hbm.at[pl.ds(base + (c - 2) * _CH, _CH)],
                    ssem[gb],
                ).wait()

            @pl.when(c + 2 < _NCH)
            def _():
                pltpu.async_copy(table_sh.at[idx_v.at[c + 2]], rows_v.at[gb],
                                 gsem[gb])

            pltpu.make_async_copy(table_sh.at[idx_v.at[c]], rows_v.at[b],
                                  gsem[b]).wait()

            p0 = lax.rem(c * _CH, _L)

            @pl.loop(0, _CH, unroll=4)
            def _row(r):
                p = p0 + r
                p = jnp.where(p >= _L, p - _L, p)
                for d in range(_D // _LANES):
                    sl = pl.ds(d * _LANES, _LANES)
                    plsc.addupdate(rows_v.at[b, r, sl], pos_v[p, sl])

            pltpu.async_copy(rows_v.at[b],
                             out_hbm.at[pl.ds(base + c * _CH, _CH)], ssem[b])

    for c in (_NCH - 2, _NCH - 1):
        pltpu.make_async_copy(rows_v.at[c % _NBUF],
                              out_hbm.at[pl.ds(base + c * _CH, _CH)],
                              ssem[c % _NBUF]).wait()


def kernel(x, start_token, end_token, embedding_table):
    idx = x.reshape(_NW, _NCH, _CH).astype(jnp.int32)
    pos = _pos_encoding()
    out = _emb_kernel(embedding_table, idx, pos)
    return out.reshape(_B, _L, _D)


# speedup vs baseline: 9.1370x; 2.2812x over previous
import functools

import jax
import jax.numpy as jnp
from jax import lax
from jax.experimental import pallas as pl
from jax.experimental.pallas import tpu as pltpu
from jax.experimental.pallas import tpu_sc as plsc

_V = 1000
_D = 128
_L = 200
_B = 4096

_N = _B * _L
_NW = 32
_SEQ_W = _B // _NW
_TOK_W = _N // _NW
_CH = _SEQ_W
_NCH = _L
_NBUF = 4
_LANES = 16


def _pos_encoding():
    even_i = jnp.arange(0, _D, 2).astype(jnp.float32)
    denominator = jnp.power(10000.0, even_i / _D)
    position = jnp.arange(_L).reshape(_L, 1).astype(jnp.float32)
    even_pos = jnp.sin(position / denominator)
    odd_pos = jnp.cos(position / denominator)
    return jnp.stack([even_pos, odd_pos], axis=2).reshape(_L, _D)


_mesh = plsc.VectorSubcoreMesh(core_axis_name="c", subcore_axis_name="s")


@functools.partial(
    pl.kernel,
    out_type=jax.ShapeDtypeStruct((_N, _D), jnp.float32),
    mesh=_mesh,
    scratch_types=[
        pltpu.VMEM((_NCH, _CH), jnp.int32),
        pltpu.VMEM((_L, _D), jnp.float32),
        pltpu.VMEM((_NBUF, _CH, _D), jnp.float32),
        pltpu.VMEM((_NBUF, _CH), jnp.int32),
        pltpu.VMEM((1, _CH), jnp.int32),
        pltpu.VMEM_SHARED((_V, _D), jnp.float32),
        [pltpu.SemaphoreType.DMA] * _NBUF,
        [pltpu.SemaphoreType.DMA] * _NBUF,
    ],
)
def _emb_kernel(table_hbm, idx_hbm, pos_hbm, out_hbm, idx_v, pos_v, rows_v,
                oidx_v, obase_v, table_sh, gsem, ssem):
    sid = lax.axis_index("s")
    wid = sid * 2 + lax.axis_index("c")

    @pl.when(sid == 0)
    def _():
        pltpu.sync_copy(table_hbm, table_sh)

    pltpu.sync_copy(idx_hbm.at[wid], idx_v)
    pltpu.sync_copy(pos_hbm, pos_v)
    base = wid * _TOK_W
    for j in range(_CH // _LANES):
        obase_v[0, pl.ds(j * _LANES, _LANES)] = (
            (lax.iota(jnp.int32, _LANES) + j * _LANES) * _L + base)
    plsc.subcore_barrier()

    def store(c, b):
        return pltpu.make_async_copy(rows_v.at[b],
                                     out_hbm.at[oidx_v.at[b]], ssem[b])

    pltpu.async_copy(table_sh.at[idx_v.at[0]], rows_v.at[0], gsem[0])
    pltpu.async_copy(table_sh.at[idx_v.at[1]], rows_v.at[1], gsem[1])

    @pl.loop(0, _NCH, step=_NBUF)
    def _chunk(c0):
        for b in range(_NBUF):
            c = c0 + b
            gb = (b + 2) % _NBUF

            @pl.when(c >= 2)
            def _():
                store(c - 2, gb).wait()

            @pl.when(c + 2 < _NCH)
            def _():
                pltpu.async_copy(table_sh.at[idx_v.at[c + 2]], rows_v.at[gb],
                                 gsem[gb])

            pltpu.make_async_copy(table_sh.at[idx_v.at[c]], rows_v.at[b],
                                  gsem[b]).wait()

            for j in range(_CH // _LANES):
                sl = pl.ds(j * _LANES, _LANES)
                oidx_v[b, sl] = obase_v[0, sl] + c

            for d in range(_D // _LANES):
                sl = pl.ds(d * _LANES, _LANES)
                pvec = pos_v[c, sl]

                @pl.loop(0, _CH, unroll=8)
                def _row(r, sl=sl, pvec=pvec):
                    plsc.addupdate(rows_v.at[b, r, sl], pvec)

            store(c, b).start()

    for c in (_NCH - 2, _NCH - 1):
        store(c, c % _NBUF).wait()


def kernel(x, start_token, end_token, embedding_table):
    idx = x.reshape(_NW, _SEQ_W, _L).transpose(0, 2, 1).astype(jnp.int32)
    pos = _pos_encoding()
    out = _emb_kernel(embedding_table, idx, pos)
    return out.reshape(_B, _L, _D)


# speedup vs baseline: 10.3860x; 1.1367x over previous
import functools

import jax
import jax.numpy as jnp
from jax import lax
from jax.experimental import pallas as pl
from jax.experimental.pallas import tpu as pltpu
from jax.experimental.pallas import tpu_sc as plsc

_V = 1000
_D = 128
_L = 200
_B = 4096

_N = _B * _L
_NW = 32
_SEQ_W = _B // _NW
_TOK_W = _N // _NW
_CH = _SEQ_W
_NCH = _L
_NBUF = 4
_LANES = 16


def _pos_encoding():
    even_i = jnp.arange(0, _D, 2).astype(jnp.float32)
    denominator = jnp.power(10000.0, even_i / _D)
    position = jnp.arange(_L).reshape(_L, 1).astype(jnp.float32)
    even_pos = jnp.sin(position / denominator)
    odd_pos = jnp.cos(position / denominator)
    return jnp.stack([even_pos, odd_pos], axis=2).reshape(_L, _D)


_mesh = plsc.VectorSubcoreMesh(core_axis_name="c", subcore_axis_name="s")


@functools.partial(
    pl.kernel,
    out_type=jax.ShapeDtypeStruct((_N, _D), jnp.float32),
    mesh=_mesh,
    scratch_types=[
        pltpu.VMEM((_NCH, _CH), jnp.int32),
        pltpu.VMEM((_L, _D), jnp.float32),
        pltpu.VMEM((_NBUF, _CH, _D), jnp.float32),
        pltpu.VMEM((_NBUF, 2, _CH // 2), jnp.int32),
        pltpu.VMEM((1, _CH), jnp.int32),
        pltpu.VMEM_SHARED((_V, _D), jnp.float32),
        [pltpu.SemaphoreType.DMA] * _NBUF,
        [pltpu.SemaphoreType.DMA] * _NBUF,
    ],
)
def _emb_kernel(table_hbm, idx_hbm, pos_hbm, out_hbm, idx_v, pos_v, rows_v,
                oidx_v, obase_v, table_sh, gsem, ssem):
    sid = lax.axis_index("s")
    wid = sid * 2 + lax.axis_index("c")

    @pl.when(sid == 0)
    def _():
        pltpu.sync_copy(table_hbm, table_sh)

    pltpu.sync_copy(idx_hbm.at[wid], idx_v)
    pltpu.sync_copy(pos_hbm, pos_v)
    base = wid * _TOK_W
    for j in range(_CH // _LANES):
        obase_v[0, pl.ds(j * _LANES, _LANES)] = (
            (lax.iota(jnp.int32, _LANES) + j * _LANES) * _L + base)
    plsc.subcore_barrier()

    def store_half(b, h):
        return pltpu.make_async_copy(
            rows_v.at[b, pl.ds(h * (_CH // 2), _CH // 2)],
            out_hbm.at[oidx_v.at[b, h]], ssem[b])

    pltpu.async_copy(table_sh.at[idx_v.at[0]], rows_v.at[0], gsem[0])
    pltpu.async_copy(table_sh.at[idx_v.at[1]], rows_v.at[1], gsem[1])

    @pl.loop(0, _NCH, step=_NBUF)
    def _chunk(c0):
        for b in range(_NBUF):
            c = c0 + b
            gb = (b + 2) % _NBUF

            @pl.when(c >= 2)
            def _():
                store_half(gb, 0).wait()
                store_half(gb, 1).wait()

            @pl.when(c + 2 < _NCH)
            def _():
                pltpu.async_copy(table_sh.at[idx_v.at[c + 2]], rows_v.at[gb],
                                 gsem[gb])

            for j in range(_CH // _LANES):
                h, jj = divmod(j, _CH // 2 // _LANES)
                sl = pl.ds(jj * _LANES, _LANES)
                oidx_v[b, h, sl] = obase_v[0, pl.ds(j * _LANES, _LANES)] + c

            pltpu.make_async_copy(table_sh.at[idx_v.at[c]], rows_v.at[b],
                                  gsem[b]).wait()

            for h in range(2):
                r0 = h * (_CH // 2)
                for d in range(_D // _LANES):
                    sl = pl.ds(d * _LANES, _LANES)
                    pvec = pos_v[c, sl]

                    @pl.loop(r0, r0 + _CH // 2, unroll=4)
                    def _row(r, sl=sl, pvec=pvec):
                        plsc.addupdate(rows_v.at[b, r, sl], pvec)

                store_half(b, h).start()

    for c in (_NCH - 2, _NCH - 1):
        for h in range(2):
            store_half(c % _NBUF, h).wait()


def kernel(x, start_token, end_token, embedding_table):
    idx = x.reshape(_NW, _SEQ_W, _L).transpose(0, 2, 1).astype(jnp.int32)
    pos = _pos_encoding()
    out = _emb_kernel(embedding_table, idx, pos)
    return out.reshape(_B, _L, _D)


# speedup vs baseline: 12.7991x; 1.2323x over previous
import functools

import jax
import jax.numpy as jnp
from jax import lax
from jax.experimental import pallas as pl
from jax.experimental.pallas import tpu as pltpu
from jax.experimental.pallas import tpu_sc as plsc

_V = 1000
_D = 128
_L = 200
_B = 4096

_N = _B * _L
_NW = 32
_SEQ_W = _B // _NW
_TOK_W = _N // _NW
_CH = _SEQ_W
_NCH = _L
_NBUF = 4
_LANES = 16


def _pos_encoding():
    even_i = jnp.arange(0, _D, 2).astype(jnp.float32)
    denominator = jnp.power(10000.0, even_i / _D)
    position = jnp.arange(_L).reshape(_L, 1).astype(jnp.float32)
    even_pos = jnp.sin(position / denominator)
    odd_pos = jnp.cos(position / denominator)
    return jnp.stack([even_pos, odd_pos], axis=2).reshape(_L, _D)


_mesh = plsc.VectorSubcoreMesh(core_axis_name="c", subcore_axis_name="s")


@functools.partial(
    pl.kernel,
    out_type=jax.ShapeDtypeStruct((_N, _D), jnp.float32),
    mesh=_mesh,
    scratch_types=[
        pltpu.VMEM((_NCH, _CH), jnp.int32),
        pltpu.VMEM((_L, _D), jnp.float32),
        pltpu.VMEM((_NBUF, _CH, _D), jnp.float32),
        pltpu.VMEM((_NBUF, 2, _CH // 2), jnp.int32),
        pltpu.VMEM((1, _CH), jnp.int32),
        pltpu.VMEM_SHARED((_V, _D), jnp.float32),
        [pltpu.SemaphoreType.DMA] * _NBUF,
        [pltpu.SemaphoreType.DMA] * _NBUF,
    ],
)
def _emb_kernel(table_hbm, idx_hbm, pos_hbm, out_hbm, idx_v, pos_v, rows_v,
                oidx_v, obase_v, table_sh, gsem, ssem):
    sid = lax.axis_index("s")
    wid = sid * 2 + lax.axis_index("c")

    @pl.when(sid == 0)
    def _():
        pltpu.sync_copy(table_hbm, table_sh)

    pltpu.sync_copy(idx_hbm.at[wid], idx_v)
    pltpu.sync_copy(pos_hbm, pos_v)
    base = wid * _TOK_W
    for j in range(_CH // _LANES):
        obase_v[0, pl.ds(j * _LANES, _LANES)] = (
            (lax.iota(jnp.int32, _LANES) + j * _LANES) * _L + base)
    plsc.subcore_barrier()

    def store_half(b, h):
        return pltpu.make_async_copy(
            rows_v.at[b, pl.ds(h * (_CH // 2), _CH // 2)],
            out_hbm.at[oidx_v.at[b, h]], ssem[b])

    pltpu.async_copy(table_sh.at[idx_v.at[0]], rows_v.at[0], gsem[0])
    pltpu.async_copy(table_sh.at[idx_v.at[1]], rows_v.at[1], gsem[1])

    @pl.loop(0, _NCH, step=_NBUF)
    def _chunk(c0):
        for b in range(_NBUF):
            c = c0 + b
            gb = (b + 2) % _NBUF

            @pl.when(c >= 2)
            def _():
                store_half(gb, 0).wait()
                store_half(gb, 1).wait()

            @pl.when(c + 2 < _NCH)
            def _():
                pltpu.async_copy(table_sh.at[idx_v.at[c + 2]], rows_v.at[gb],
                                 gsem[gb])

            for j in range(_CH // _LANES):
                h, jj = divmod(j, _CH // 2 // _LANES)
                sl = pl.ds(jj * _LANES, _LANES)
                oidx_v[b, h, sl] = obase_v[0, pl.ds(j * _LANES, _LANES)] + c

            pltpu.make_async_copy(table_sh.at[idx_v.at[c]], rows_v.at[b],
                                  gsem[b]).wait()

            pvecs = [pos_v[c, pl.ds(d * _LANES, _LANES)]
                     for d in range(_D // _LANES)]
            for h in range(2):
                r0 = h * (_CH // 2)

                @pl.loop(r0, r0 + _CH // 2, unroll=4)
                def _row(r):
                    for d in range(_D // _LANES):
                        plsc.addupdate(
                            rows_v.at[b, r, pl.ds(d * _LANES, _LANES)],
                            pvecs[d])

                store_half(b, h).start()

    for c in (_NCH - 2, _NCH - 1):
        for h in range(2):
            store_half(c % _NBUF, h).wait()


def kernel(x, start_token, end_token, embedding_table):
    idx = x.reshape(_NW, _SEQ_W, _L).transpose(0, 2, 1).astype(jnp.int32)
    pos = _pos_encoding()
    out = _emb_kernel(embedding_table, idx, pos)
    return out.reshape(_B, _L, _D)
